# Initial kernel scaffold; baseline (speedup 1.0000x reference)
import jax
import jax.numpy as jnp
from jax.experimental import pallas as pl


def kernel(loc_data, conf_data, priors, seg_data, loc_t, conf_t, segs):
    raise NotImplementedError("write your pallas kernel here")


# speedup vs baseline: 4.8443x; 4.8443x over previous
import jax
import jax.numpy as jnp
from jax import lax
from jax.experimental import pallas as pl
from jax.experimental.pallas import tpu as pltpu


def _mbl_kernel(conf_ref, ct_ref, locd_ref, loct_ref, sd_ref, sg_ref,
                out_ref, m_s, np_s, acc_s):
    i = pl.program_id(0)
    num = pl.num_programs(0)
    x = conf_ref[0]
    t = ct_ref[0]
    nc, d = x.shape

    s = jnp.sum(jnp.exp(x), axis=0, keepdims=True)
    lse = jnp.log(s)
    cls = lax.broadcasted_iota(jnp.int32, (nc, d), 0)
    g = jnp.sum(jnp.where(cls == t, x, 0.0), axis=0, keepdims=True)
    ce = lse - g
    pos = t > 0
    posf = pos.astype(jnp.float32)
    m = jnp.maximum(jnp.where(pos, 0.0, ce), 0.0)
    m_s[pl.ds(i, 1)] = m[None]

    npos = jnp.sum(posf)
    np_s[pl.ds(i, 1)] = jnp.full((1, 1, 128), npos, jnp.float32)

    dd = locd_ref[0] - loct_ref[0]
    ad = jnp.abs(dd)
    sl1 = jnp.where(ad < 1.0, 0.5 * dd * dd, ad - 0.5)
    sl1_sum = jnp.sum(sl1 * posf)
    posce = jnp.sum(ce * posf)

    sdv = sd_ref[0]
    sgv = sg_ref[0]
    inter = jnp.sum(sdv * sgv)
    union = jnp.sum(sdv + sgv)

    @pl.when(i == 0)
    def _init():
        acc_s[0] = 0.0
        acc_s[1] = 0.0
        acc_s[2] = 0.0
        acc_s[3] = 0.0

    acc_s[0] = acc_s[0] + sl1_sum
    acc_s[1] = acc_s[1] + posce
    acc_s[2] = acc_s[2] + inter
    acc_s[3] = acc_s[3] + union

    @pl.when(i == num - 1)
    def _final():
        mm = m_s[...]
        bits = lax.bitcast_convert_type(mm, jnp.int32)
        npos_v = np_s[:, :, 0:1]
        k = jnp.minimum(3 * npos_v.astype(jnp.int32), d - 1)

        def body(_, carry):
            lo, hi = carry
            mid = lo + (hi - lo) // 2
            cnt = jnp.sum((bits > mid).astype(jnp.int32), axis=2,
                          keepdims=True)
            shrink = cnt < k
            return (jnp.where(shrink, lo, mid + 1),
                    jnp.where(shrink, mid, hi))

        lo0 = jnp.zeros((num, 1, 1), jnp.int32)
        hi0 = jnp.full((num, 1, 1), jnp.int32(0x7F800000))
        lo, _ = lax.fori_loop(0, 31, body, (lo0, hi0))
        gt = bits > lo
        cnt_gt = jnp.sum(gt.astype(jnp.int32), axis=2, keepdims=True)
        sum_gt = jnp.sum(jnp.where(gt, mm, 0.0), axis=2, keepdims=True)
        tf = jnp.where(k > 0, lax.bitcast_convert_type(lo, jnp.float32), 0.0)
        topk = sum_gt + (k - cnt_gt).astype(jnp.float32) * tf
        topk_total = jnp.sum(topk)
        n_tot = jnp.sum(npos_v)
        out_ref[0] = acc_s[0] / n_tot
        out_ref[1] = (acc_s[1] + topk_total) / n_tot
        out_ref[2] = 1.0 - 2.0 * acc_s[2] / (acc_s[3] + 1e-5)


def kernel(loc_data, conf_data, priors, seg_data, loc_t, conf_t, segs):
    num, p, a, nc = conf_data.shape
    d = p * a
    seg_n = segs.shape[1]
    conf_tr = jnp.swapaxes(conf_data.reshape(num, d, nc), 1, 2)
    locd_tr = jnp.swapaxes(loc_data.reshape(num, d, 4), 1, 2)
    loct_tr = jnp.swapaxes(loc_t.reshape(num, d, 4), 1, 2)
    ct = conf_t.reshape(num, 1, d).astype(jnp.int32)
    sd = seg_data.reshape(num, 1, seg_n)
    sg = segs.reshape(num, 1, seg_n)

    out = pl.pallas_call(
        _mbl_kernel,
        grid=(num,),
        in_specs=[
            pl.BlockSpec((1, nc, d), lambda i: (i, 0, 0)),
            pl.BlockSpec((1, 1, d), lambda i: (i, 0, 0)),
            pl.BlockSpec((1, 4, d), lambda i: (i, 0, 0)),
            pl.BlockSpec((1, 4, d), lambda i: (i, 0, 0)),
            pl.BlockSpec((1, 1, seg_n), lambda i: (i, 0, 0)),
            pl.BlockSpec((1, 1, seg_n), lambda i: (i, 0, 0)),
        ],
        out_specs=pl.BlockSpec(memory_space=pltpu.SMEM),
        out_shape=jax.ShapeDtypeStruct((4,), jnp.float32),
        scratch_shapes=[
            pltpu.VMEM((num, 1, d), jnp.float32),
            pltpu.VMEM((num, 1, 128), jnp.float32),
            pltpu.SMEM((4,), jnp.float32),
        ],
    )(conf_tr, ct, locd_tr, loct_tr, sd, sg)
    return (out[0], out[1], out[2])
